# Initial kernel scaffold; baseline (speedup 1.0000x reference)
import jax
import jax.numpy as jnp
from jax.experimental import pallas as pl


def kernel(x, table):
    raise NotImplementedError("write your pallas kernel here")


# speedup vs baseline: 2.1527x; 2.1527x over previous
import functools

import numpy as np
import jax
import jax.numpy as jnp
from jax import lax
from jax.experimental import pallas as pl
from jax.experimental.pallas import tpu as pltpu
from jax.experimental.pallas import tpu_sc as plsc

WINDOW_SIZE = 100000
E = 64
B = 4096
S = 200
SCALE = 8.0

NC = 2
NS = 16
NW = NC * NS
ROWS_PER_W = (B * S) // NW
CHUNK = 80
NCHUNK = ROWS_PER_W // CHUNK
POS_PERIOD = S // np.gcd(CHUNK, S)
POS_EXT = 160 + CHUNK


def _positional_encoding() -> np.ndarray:
    half = E // 2
    positions = np.arange(S, dtype=np.float32)[:, None]
    depths = np.arange(half, dtype=np.float32)[None, :] / float(half)
    angle_rads = positions * (1.0 / (10000.0 ** depths))
    return np.concatenate(
        [np.sin(angle_rads), np.cos(angle_rads)], axis=-1
    ).astype(np.float32)


_POS = _positional_encoding()


_MESH = plsc.VectorSubcoreMesh(core_axis_name="c", subcore_axis_name="s")


@functools.partial(
    pl.kernel,
    mesh=_MESH,
    compiler_params=pltpu.CompilerParams(use_tc_tiling_on_sc=False),
    out_type=jax.ShapeDtypeStruct((B * S, E), jnp.float32),
    scratch_types=[
        pltpu.VMEM((ROWS_PER_W,), jnp.int32),
        pltpu.VMEM((POS_EXT, E), jnp.float32),
        pltpu.VMEM((CHUNK, E), jnp.float32),
        pltpu.SemaphoreType.DMA,
    ],
)
def _embed_pos(x_hbm, table_hbm, pos_hbm, out_hbm, idx_v, pos_v, rows_v, sem):
    wid = lax.axis_index("s") * NC + lax.axis_index("c")
    base = wid * ROWS_PER_W
    pltpu.sync_copy(x_hbm.at[pl.ds(base, ROWS_PER_W)], idx_v)
    pltpu.sync_copy(pos_hbm, pos_v)

    def chunk_body(c, carry):
        pltpu.async_copy(
            table_hbm.at[idx_v.at[pl.ds(c * CHUNK, CHUNK)]], rows_v, sem
        ).wait()
        off = lax.rem(c * CHUNK, S)

        def s_body(s, carry2):
            for j in range(E // 16):
                sl = pl.ds(j * 16, 16)
                r = rows_v[s, sl]
                p = pos_v[off + s, sl]
                rows_v[s, sl] = r * SCALE + p
            return carry2

        lax.fori_loop(0, CHUNK, s_body, 0, unroll=2)
        pltpu.sync_copy(rows_v, out_hbm.at[pl.ds(base + c * CHUNK, CHUNK)])
        return carry

    lax.fori_loop(0, NCHUNK, chunk_body, 0)


@jax.jit
def _run(x, table):
    xf = x.reshape(-1).astype(jnp.int32)
    pos_ext = jnp.asarray(np.concatenate([_POS, _POS[: POS_EXT - S]], axis=0))
    out = _embed_pos(xf, table, pos_ext)
    return out.reshape(B, S, E)


def kernel(x, table):
    return _run(x, table)


# speedup vs baseline: 2.8978x; 1.3461x over previous
import functools

import numpy as np
import jax
import jax.numpy as jnp
from jax import lax
from jax.experimental import pallas as pl
from jax.experimental.pallas import tpu as pltpu
from jax.experimental.pallas import tpu_sc as plsc

WINDOW_SIZE = 100000
E = 64
B = 4096
S = 200
SCALE = 8.0

NC = 2
NS = 16
NW = NC * NS
ROWS_PER_W = (B * S) // NW
CHUNK = 80
NCHUNK = ROWS_PER_W // CHUNK
POS_PERIOD = S // np.gcd(CHUNK, S)
POS_EXT = 160 + CHUNK


def _positional_encoding() -> np.ndarray:
    half = E // 2
    positions = np.arange(S, dtype=np.float32)[:, None]
    depths = np.arange(half, dtype=np.float32)[None, :] / float(half)
    angle_rads = positions * (1.0 / (10000.0 ** depths))
    return np.concatenate(
        [np.sin(angle_rads), np.cos(angle_rads)], axis=-1
    ).astype(np.float32)


_POS = _positional_encoding()


_MESH = plsc.VectorSubcoreMesh(core_axis_name="c", subcore_axis_name="s")


NBUF = 4


@functools.partial(
    pl.kernel,
    mesh=_MESH,
    compiler_params=pltpu.CompilerParams(use_tc_tiling_on_sc=False),
    out_type=jax.ShapeDtypeStruct((B * S, E), jnp.float32),
    scratch_types=[
        pltpu.VMEM((ROWS_PER_W,), jnp.int32),
        pltpu.VMEM((POS_EXT, E), jnp.float32),
    ]
    + [pltpu.VMEM((CHUNK, E), jnp.float32) for _ in range(NBUF)]
    + [pltpu.SemaphoreType.DMA for _ in range(2 * NBUF)],
)
def _embed_pos(x_hbm, table_hbm, pos_hbm, out_hbm, idx_v, pos_v, *bufs_sems):
    rbufs = bufs_sems[:NBUF]
    gsems = bufs_sems[NBUF : 2 * NBUF]
    ssems = bufs_sems[2 * NBUF :]
    wid = lax.axis_index("s") * NC + lax.axis_index("c")
    base = wid * ROWS_PER_W
    pltpu.sync_copy(x_hbm.at[pl.ds(base, ROWS_PER_W)], idx_v)
    pltpu.sync_copy(pos_hbm, pos_v)

    def gather_start(c, b):
        pltpu.async_copy(
            table_hbm.at[idx_v.at[pl.ds(c * CHUNK, CHUNK)]], rbufs[b], gsems[b]
        )

    for b in range(NBUF):
        gather_start(b, b)

    def outer(c0, carry):
        for b in range(NBUF):
            c = c0 * NBUF + b
            rows = rbufs[b]
            pltpu.make_async_copy(
                table_hbm.at[idx_v.at[pl.ds(0, CHUNK)]], rows, gsems[b]
            ).wait()
            off = lax.rem(c * CHUNK, S)

            def s_body(s, carry2):
                for j in range(E // 16):
                    sl = pl.ds(j * 16, 16)
                    rows[s, sl] = rows[s, sl] * SCALE + pos_v[off + s, sl]
                return carry2

            lax.fori_loop(0, CHUNK, s_body, 0, unroll=2)

            pb = (b - 1) % NBUF

            @pl.when(c > 0)
            def _():
                pltpu.make_async_copy(
                    rbufs[pb], out_hbm.at[pl.ds(0, CHUNK)], ssems[pb]
                ).wait()

            @pl.when((c > 0) & (c - 1 + NBUF < NCHUNK))
            def _():
                gather_start(c - 1 + NBUF, pb)

            pltpu.async_copy(
                rows, out_hbm.at[pl.ds(base + c * CHUNK, CHUNK)], ssems[b]
            )
        return carry

    lax.fori_loop(0, NCHUNK // NBUF, outer, 0)
    lb = (NCHUNK - 1) % NBUF
    pltpu.make_async_copy(
        rbufs[lb], out_hbm.at[pl.ds(0, CHUNK)], ssems[lb]
    ).wait()


@jax.jit
def _run(x, table):
    xf = x.reshape(-1).astype(jnp.int32)
    pos_ext = jnp.asarray(np.concatenate([_POS, _POS[: POS_EXT - S]], axis=0))
    out = _embed_pos(xf, table, pos_ext)
    return out.reshape(B, S, E)


def kernel(x, table):
    return _run(x, table)
